# baseline (speedup 1.0000x reference)
import functools

import jax
import jax.numpy as jnp
from jax import lax
from jax.experimental import pallas as pl
from jax.experimental.pallas import tpu as pltpu
from jax.experimental.pallas import tpu_sc as plsc

ALPHA = 16.0
N = 4096
L = 16
NW = 32
RW = N // NW
NCH = N // L
LN2 = 0.6931471805599453
SQRT2 = 1.4142135


def _vlog(x):
    bits = lax.bitcast_convert_type(x, jnp.int32)
    e = ((bits >> 23) & 0xFF) - 127
    m = lax.bitcast_convert_type((bits & 0x007FFFFF) | 0x3F800000, jnp.float32)
    big = m > SQRT2
    m2 = jnp.where(big, m * 0.5, m)
    e2 = (e + jnp.where(big, 1, 0)).astype(jnp.float32)
    z = (m2 - 1.0) / (m2 + 1.0)
    z2 = z * z
    poly = 1.0 + z2 * ((1.0 / 3.0) + z2 * (0.2 + z2 * (1.0 / 7.0)))
    return e2 * LN2 + 2.0 * z * poly


def _splat(v, dtype=jnp.float32):
    return jnp.full((L,), v, dtype)


def _row_pass(x_v, t_v, xi, ti, thr, unroll):
    def body(c, carry):
        p_acc, s_acc = carry
        xs = x_v[pl.ds(c * L, L)]
        ts = t_v[pl.ds(c * L, L)]
        sim = jnp.abs(xs - xi)
        pos = ts == ti
        lt1 = sim < 1.0
        below = sim < thr
        w = jnp.exp(sim * (-ALPHA))
        pn = (pos & lt1) & below
        selb = below & (jnp.logical_not(pos) | lt1)
        p_acc = p_acc + jnp.where(pn, w, 0.0)
        s_acc = s_acc + jnp.where(selb, w, 0.0)
        return p_acc, s_acc

    zero = jnp.zeros((L,), jnp.float32)
    return lax.fori_loop(0, NCH, body, (zero, zero), unroll=unroll)


def _sc_body(x_hbm, t_hbm, loss_hbm, prec_hbm, stats_hbm,
             x_v, t_v, lo_v, pr_v, st_v):
    wid = lax.axis_index("s") * 2 + lax.axis_index("c")
    pltpu.sync_copy(x_hbm, x_v.at[pl.ds(0, N)])
    pltpu.sync_copy(t_hbm, t_v.at[pl.ds(0, N)])
    row0 = wid * RW

    def row_body(r, carry):
        loss_acc, prec_acc = carry
        row = row0 + r
        xc = x_v[pl.ds(row, L)]
        tc = t_v[pl.ds(row, L)]
        xi = jnp.full((L,), xc[0], jnp.float32)
        ti = jnp.full((L,), tc[0], jnp.int32)

        def max_body(c, m):
            xs = x_v[pl.ds(c * L, L)]
            ts = t_v[pl.ds(c * L, L)]
            sim = jnp.abs(xs - xi)
            excl = (ts == ti) & (sim >= 1.0)
            return jnp.maximum(m, jnp.where(excl, -1.0, sim))

        m = lax.fori_loop(0, NCH, max_body, jnp.full((L,), -1.0, jnp.float32),
                          unroll=16)
        thr = _splat(jnp.max(m))

        p_acc, s_acc = _row_pass(x_v, t_v, xi, ti, thr, unroll=16)
        pv = _splat(jnp.sum(p_acc))
        sv = _splat(jnp.sum(s_acc))
        ratio = pv / sv
        loss_v = jnp.where(pv > 0.0, -_vlog(ratio), 0.0)
        loss_acc = loss_acc + loss_v
        prec_acc = prec_acc + jnp.where(loss_v < 0.6, 1.0, 0.0)
        return loss_acc, prec_acc

    zero = jnp.zeros((L,), jnp.float32)
    loss_acc, prec_acc = lax.fori_loop(0, RW, row_body, (zero, zero))

    lo_v[...] = loss_acc
    pr_v[...] = prec_acc
    pltpu.sync_copy(lo_v, loss_hbm.at[wid])
    pltpu.sync_copy(pr_v, prec_hbm.at[wid])

    @pl.when(wid == NW - 1)
    def _stats():
        xc = x_v[pl.ds(N - 1, L)]
        tc = t_v[pl.ds(N - 1, L)]
        xi = jnp.full((L,), xc[0], jnp.float32)
        ti = jnp.full((L,), tc[0], jnp.int32)

        def st_body(c, carry):
            sp, cp, sn, cn = carry
            xs = x_v[pl.ds(c * L, L)]
            ts = t_v[pl.ds(c * L, L)]
            sim = jnp.abs(xs - xi)
            pos = ts == ti
            lp = pos & (sim < 1.0)
            sp = sp + jnp.where(lp, sim, 0.0)
            cp = cp + jnp.where(lp, 1.0, 0.0)
            sn = sn + jnp.where(pos, 0.0, sim)
            cn = cn + jnp.where(pos, 0.0, 1.0)
            return sp, cp, sn, cn

        z = jnp.zeros((L,), jnp.float32)
        sp, cp, sn, cn = lax.fori_loop(0, NCH, st_body, (z, z, z, z), unroll=16)
        st_v[0, :] = _splat(jnp.sum(sp)) / _splat(jnp.sum(cp))
        st_v[1, :] = _splat(jnp.sum(sn)) / _splat(jnp.sum(cn))
        pltpu.sync_copy(st_v, stats_hbm)


_sc_call = functools.partial(
    pl.kernel,
    mesh=plsc.VectorSubcoreMesh(core_axis_name="c", subcore_axis_name="s"),
    compiler_params=pltpu.CompilerParams(needs_layout_passes=False),
    out_type=[
        jax.ShapeDtypeStruct((NW, L), jnp.float32),
        jax.ShapeDtypeStruct((NW, L), jnp.float32),
        jax.ShapeDtypeStruct((2, L), jnp.float32),
    ],
    scratch_types=[
        pltpu.VMEM((N + L,), jnp.float32),
        pltpu.VMEM((N + L,), jnp.int32),
        pltpu.VMEM((L,), jnp.float32),
        pltpu.VMEM((L,), jnp.float32),
        pltpu.VMEM((2, L), jnp.float32),
    ],
)(_sc_body)


def kernel(inputs, targets):
    t32 = targets.astype(jnp.int32)
    loss_p, prec_p, stats = _sc_call(inputs, t32)
    loss = jnp.sum(loss_p[:, 0]) * (1.0 / N)
    prec = jnp.sum(prec_p[:, 0]) * (1.0 / N)
    return (loss, prec, stats[0, 0], stats[1, 0])


# speedup vs baseline: 3.1153x; 3.1153x over previous
import functools

import jax
import jax.numpy as jnp
from jax import lax
from jax.experimental import pallas as pl
from jax.experimental.pallas import tpu as pltpu
from jax.experimental.pallas import tpu_sc as plsc

ALPHA = 16.0
N = 4096
L = 16
NW = 32
RW = N // NW
NCH = N // L
LN2 = 0.6931471805599453
SQRT2 = 1.4142135


def _vlog(x):
    bits = lax.bitcast_convert_type(x, jnp.int32)
    e = ((bits >> 23) & 0xFF) - 127
    m = lax.bitcast_convert_type((bits & 0x007FFFFF) | 0x3F800000, jnp.float32)
    big = m > SQRT2
    m2 = jnp.where(big, m * 0.5, m)
    e2 = (e + jnp.where(big, 1, 0)).astype(jnp.float32)
    z = (m2 - 1.0) / (m2 + 1.0)
    z2 = z * z
    poly = 1.0 + z2 * ((1.0 / 3.0) + z2 * (0.2 + z2 * (1.0 / 7.0)))
    return e2 * LN2 + 2.0 * z * poly


def _splat(v, dtype=jnp.float32):
    return jnp.full((L,), v, dtype)


def _row_pass(x_v, t_v, xi, ti, thr, unroll):
    def body(c, carry):
        p_acc, s_acc = carry
        xs = x_v[pl.ds(c * L, L)]
        ts = t_v[pl.ds(c * L, L)]
        sim = jnp.abs(xs - xi)
        pos = ts == ti
        lt1 = sim < 1.0
        below = sim < thr
        w = jnp.exp(sim * (-ALPHA))
        pn = (pos & lt1) & below
        selb = below & (jnp.logical_not(pos) | lt1)
        p_acc = p_acc + jnp.where(pn, w, 0.0)
        s_acc = s_acc + jnp.where(selb, w, 0.0)
        return p_acc, s_acc

    zero = jnp.zeros((L,), jnp.float32)
    return lax.fori_loop(0, NCH, body, (zero, zero), unroll=unroll)


def _sc_body(x_hbm, t_hbm, loss_hbm, prec_hbm, stats_hbm,
             x_v, t_v, lo_v, pr_v, st_v):
    wid = lax.axis_index("s") * 2 + lax.axis_index("c")
    pltpu.sync_copy(x_hbm, x_v.at[pl.ds(0, N)])
    pltpu.sync_copy(t_hbm, t_v.at[pl.ds(0, N)])
    row0 = wid * RW

    def row_body(r, carry):
        loss_acc, prec_acc = carry
        row = row0 + r
        xc = x_v[pl.ds(row, L)]
        tc = t_v[pl.ds(row, L)]
        xi = jnp.full((L,), xc[0], jnp.float32)
        ti = jnp.full((L,), tc[0], jnp.int32)

        def max_body(c, m):
            xs = x_v[pl.ds(c * L, L)]
            ts = t_v[pl.ds(c * L, L)]
            sim = jnp.abs(xs - xi)
            excl = (ts == ti) & (sim >= 1.0)
            return jnp.maximum(m, jnp.where(excl, -1.0, sim))

        m = lax.fori_loop(0, NCH, max_body, jnp.full((L,), -1.0, jnp.float32),
                          unroll=4)
        thr = _splat(jnp.max(m))

        p_acc, s_acc = _row_pass(x_v, t_v, xi, ti, thr, unroll=4)
        pv = _splat(jnp.sum(p_acc))
        sv = _splat(jnp.sum(s_acc))
        ratio = pv / sv
        loss_v = jnp.where(pv > 0.0, -_vlog(ratio), 0.0)
        loss_acc = loss_acc + loss_v
        prec_acc = prec_acc + jnp.where(loss_v < 0.6, 1.0, 0.0)
        return loss_acc, prec_acc

    zero = jnp.zeros((L,), jnp.float32)
    loss_acc, prec_acc = lax.fori_loop(0, RW, row_body, (zero, zero))

    lo_v[...] = loss_acc
    pr_v[...] = prec_acc
    pltpu.sync_copy(lo_v, loss_hbm.at[wid])
    pltpu.sync_copy(pr_v, prec_hbm.at[wid])

    @pl.when(wid == NW - 1)
    def _stats():
        xc = x_v[pl.ds(N - 1, L)]
        tc = t_v[pl.ds(N - 1, L)]
        xi = jnp.full((L,), xc[0], jnp.float32)
        ti = jnp.full((L,), tc[0], jnp.int32)

        def st_body(c, carry):
            sp, cp, sn, cn = carry
            xs = x_v[pl.ds(c * L, L)]
            ts = t_v[pl.ds(c * L, L)]
            sim = jnp.abs(xs - xi)
            pos = ts == ti
            lp = pos & (sim < 1.0)
            sp = sp + jnp.where(lp, sim, 0.0)
            cp = cp + jnp.where(lp, 1.0, 0.0)
            sn = sn + jnp.where(pos, 0.0, sim)
            cn = cn + jnp.where(pos, 0.0, 1.0)
            return sp, cp, sn, cn

        z = jnp.zeros((L,), jnp.float32)
        sp, cp, sn, cn = lax.fori_loop(0, NCH, st_body, (z, z, z, z), unroll=4)
        st_v[0, :] = _splat(jnp.sum(sp)) / _splat(jnp.sum(cp))
        st_v[1, :] = _splat(jnp.sum(sn)) / _splat(jnp.sum(cn))
        pltpu.sync_copy(st_v, stats_hbm)


_sc_call = functools.partial(
    pl.kernel,
    mesh=plsc.VectorSubcoreMesh(core_axis_name="c", subcore_axis_name="s"),
    compiler_params=pltpu.CompilerParams(needs_layout_passes=False),
    out_type=[
        jax.ShapeDtypeStruct((NW, L), jnp.float32),
        jax.ShapeDtypeStruct((NW, L), jnp.float32),
        jax.ShapeDtypeStruct((2, L), jnp.float32),
    ],
    scratch_types=[
        pltpu.VMEM((N + L,), jnp.float32),
        pltpu.VMEM((N + L,), jnp.int32),
        pltpu.VMEM((L,), jnp.float32),
        pltpu.VMEM((L,), jnp.float32),
        pltpu.VMEM((2, L), jnp.float32),
    ],
)(_sc_body)


def kernel(inputs, targets):
    t32 = targets.astype(jnp.int32)
    loss_p, prec_p, stats = _sc_call(inputs, t32)
    loss = jnp.sum(loss_p[:, 0]) * (1.0 / N)
    prec = jnp.sum(prec_p[:, 0]) * (1.0 / N)
    return (loss, prec, stats[0, 0], stats[1, 0])


# speedup vs baseline: 3.1908x; 1.0243x over previous
import functools

import jax
import jax.numpy as jnp
from jax import lax
from jax.experimental import pallas as pl
from jax.experimental.pallas import tpu as pltpu
from jax.experimental.pallas import tpu_sc as plsc

ALPHA = 16.0
N = 4096
L = 16
NW = 32
RW = N // NW
NCH = N // L
LN2 = 0.6931471805599453
SQRT2 = 1.4142135


def _vlog(x):
    bits = lax.bitcast_convert_type(x, jnp.int32)
    e = ((bits >> 23) & 0xFF) - 127
    m = lax.bitcast_convert_type((bits & 0x007FFFFF) | 0x3F800000, jnp.float32)
    big = m > SQRT2
    m2 = jnp.where(big, m * 0.5, m)
    e2 = (e + jnp.where(big, 1, 0)).astype(jnp.float32)
    z = (m2 - 1.0) / (m2 + 1.0)
    z2 = z * z
    poly = 1.0 + z2 * ((1.0 / 3.0) + z2 * (0.2 + z2 * (1.0 / 7.0)))
    return e2 * LN2 + 2.0 * z * poly


def _splat(v, dtype=jnp.float32):
    return jnp.full((L,), v, dtype)


def _row_pass(x_v, t_v, xi, ti, thr, unroll):
    def body(c, carry):
        p_acc, s_acc = carry
        xs = x_v[pl.ds(c * L, L)]
        ts = t_v[pl.ds(c * L, L)]
        sim = jnp.abs(xs - xi)
        pos = ts == ti
        lt1 = sim < 1.0
        below = sim < thr
        w = jnp.exp(sim * (-ALPHA))
        pn = (pos & lt1) & below
        selb = below & (jnp.logical_not(pos) | lt1)
        p_acc = p_acc + jnp.where(pn, w, 0.0)
        s_acc = s_acc + jnp.where(selb, w, 0.0)
        return p_acc, s_acc

    zero = jnp.zeros((L,), jnp.float32)
    return lax.fori_loop(0, NCH, body, (zero, zero), unroll=unroll)


def _sc_body(x_hbm, t_hbm, loss_hbm, prec_hbm, stats_hbm,
             x_v, t_v, lo_v, pr_v, st_v):
    wid = lax.axis_index("s") * 2 + lax.axis_index("c")
    pltpu.sync_copy(x_hbm, x_v.at[pl.ds(0, N)])
    pltpu.sync_copy(t_hbm, t_v.at[pl.ds(0, N)])
    row0 = wid * RW

    def row_body(r, carry):
        loss_acc, prec_acc = carry
        row = row0 + r
        xc = x_v[pl.ds(row, L)]
        tc = t_v[pl.ds(row, L)]
        xi = jnp.full((L,), xc[0], jnp.float32)
        ti = jnp.full((L,), tc[0], jnp.int32)

        def max_body(c, m):
            xs = x_v[pl.ds(c * L, L)]
            ts = t_v[pl.ds(c * L, L)]
            sim = jnp.abs(xs - xi)
            excl = (ts == ti) & (sim >= 1.0)
            return jnp.maximum(m, jnp.where(excl, -1.0, sim))

        m = lax.fori_loop(0, NCH, max_body, jnp.full((L,), -1.0, jnp.float32),
                          unroll=2)
        thr = _splat(jnp.max(m))

        p_acc, s_acc = _row_pass(x_v, t_v, xi, ti, thr, unroll=2)
        pv = _splat(jnp.sum(p_acc))
        sv = _splat(jnp.sum(s_acc))
        ratio = pv / sv
        loss_v = jnp.where(pv > 0.0, -_vlog(ratio), 0.0)
        loss_acc = loss_acc + loss_v
        prec_acc = prec_acc + jnp.where(loss_v < 0.6, 1.0, 0.0)
        return loss_acc, prec_acc

    zero = jnp.zeros((L,), jnp.float32)
    loss_acc, prec_acc = lax.fori_loop(0, RW, row_body, (zero, zero))

    lo_v[...] = loss_acc
    pr_v[...] = prec_acc
    pltpu.sync_copy(lo_v, loss_hbm.at[wid])
    pltpu.sync_copy(pr_v, prec_hbm.at[wid])

    @pl.when(wid == NW - 1)
    def _stats():
        xc = x_v[pl.ds(N - 1, L)]
        tc = t_v[pl.ds(N - 1, L)]
        xi = jnp.full((L,), xc[0], jnp.float32)
        ti = jnp.full((L,), tc[0], jnp.int32)

        def st_body(c, carry):
            sp, cp, sn, cn = carry
            xs = x_v[pl.ds(c * L, L)]
            ts = t_v[pl.ds(c * L, L)]
            sim = jnp.abs(xs - xi)
            pos = ts == ti
            lp = pos & (sim < 1.0)
            sp = sp + jnp.where(lp, sim, 0.0)
            cp = cp + jnp.where(lp, 1.0, 0.0)
            sn = sn + jnp.where(pos, 0.0, sim)
            cn = cn + jnp.where(pos, 0.0, 1.0)
            return sp, cp, sn, cn

        z = jnp.zeros((L,), jnp.float32)
        sp, cp, sn, cn = lax.fori_loop(0, NCH, st_body, (z, z, z, z), unroll=2)
        st_v[0, :] = _splat(jnp.sum(sp)) / _splat(jnp.sum(cp))
        st_v[1, :] = _splat(jnp.sum(sn)) / _splat(jnp.sum(cn))
        pltpu.sync_copy(st_v, stats_hbm)


_sc_call = functools.partial(
    pl.kernel,
    mesh=plsc.VectorSubcoreMesh(core_axis_name="c", subcore_axis_name="s"),
    compiler_params=pltpu.CompilerParams(needs_layout_passes=False),
    out_type=[
        jax.ShapeDtypeStruct((NW, L), jnp.float32),
        jax.ShapeDtypeStruct((NW, L), jnp.float32),
        jax.ShapeDtypeStruct((2, L), jnp.float32),
    ],
    scratch_types=[
        pltpu.VMEM((N + L,), jnp.float32),
        pltpu.VMEM((N + L,), jnp.int32),
        pltpu.VMEM((L,), jnp.float32),
        pltpu.VMEM((L,), jnp.float32),
        pltpu.VMEM((2, L), jnp.float32),
    ],
)(_sc_body)


def kernel(inputs, targets):
    t32 = targets.astype(jnp.int32)
    loss_p, prec_p, stats = _sc_call(inputs, t32)
    loss = jnp.sum(loss_p[:, 0]) * (1.0 / N)
    prec = jnp.sum(prec_p[:, 0]) * (1.0 / N)
    return (loss, prec, stats[0, 0], stats[1, 0])


# speedup vs baseline: 8.2641x; 2.5900x over previous
import functools

import jax
import jax.numpy as jnp
from jax import lax
from jax.experimental import pallas as pl
from jax.experimental.pallas import tpu as pltpu
from jax.experimental.pallas import tpu_sc as plsc

ALPHA = 16.0
N = 4096
K = 512
NT = N - K
R = 512
G = NT // R
L = 16
NW = 32
RW = K // NW
NCH = N // L
LN2 = 0.6931471805599453
SQRT2 = 1.4142135


def _tc_body(x_row_ref, t_row_ref, x_col_ref, t_col_ref, loss_ref, prec_ref):
    i = pl.program_id(0)

    x_row = x_row_ref[...]
    t_row = t_row_ref[...]
    x_col = x_col_ref[...]
    t_col = t_col_ref[...]

    sim = jnp.abs(x_col - x_row)
    pos = t_col == t_row
    lt1 = sim < 1.0
    excl = pos & jnp.logical_not(lt1)
    thr = jnp.max(jnp.where(excl, -1.0, sim), axis=1, keepdims=True)

    below = sim < thr
    w = jnp.exp(-ALPHA * sim)
    pn_m = (pos & lt1) & below
    selb = below & jnp.logical_not(excl)
    p = jnp.sum(jnp.where(pn_m, w, 0.0), axis=1, keepdims=True)
    s = jnp.sum(jnp.where(selb, w, 0.0), axis=1, keepdims=True)
    q = s - p

    loss_i = jnp.where(p > 0.0, -jnp.log(p / (p + q)), 0.0)

    @pl.when(i == 0)
    def _init():
        loss_ref[...] = jnp.zeros_like(loss_ref)
        prec_ref[...] = jnp.zeros_like(prec_ref)

    loss_ref[...] += jnp.sum(loss_i).reshape(1, 1)
    prec_ref[...] += jnp.sum(jnp.where(loss_i < 0.6, 1.0, 0.0)).reshape(1, 1)


def _tc_call(x_rows, t_rows, x_cols, t_cols):
    return pl.pallas_call(
        _tc_body,
        grid=(G,),
        in_specs=[
            pl.BlockSpec((R, 1), lambda i: (i, 0)),
            pl.BlockSpec((R, 1), lambda i: (i, 0)),
            pl.BlockSpec((1, N), lambda i: (0, 0)),
            pl.BlockSpec((1, N), lambda i: (0, 0)),
        ],
        out_specs=[
            pl.BlockSpec((1, 1), lambda i: (0, 0)),
            pl.BlockSpec((1, 1), lambda i: (0, 0)),
        ],
        out_shape=[jax.ShapeDtypeStruct((1, 1), jnp.float32)] * 2,
    )(x_rows, t_rows, x_cols, t_cols)


def _vlog(x):
    bits = lax.bitcast_convert_type(x, jnp.int32)
    e = ((bits >> 23) & 0xFF) - 127
    m = lax.bitcast_convert_type((bits & 0x007FFFFF) | 0x3F800000, jnp.float32)
    big = m > SQRT2
    m2 = jnp.where(big, m * 0.5, m)
    e2 = (e + jnp.where(big, 1, 0)).astype(jnp.float32)
    z = (m2 - 1.0) / (m2 + 1.0)
    z2 = z * z
    poly = 1.0 + z2 * ((1.0 / 3.0) + z2 * (0.2 + z2 * (1.0 / 7.0)))
    return e2 * LN2 + 2.0 * z * poly


def _splat(v, dtype=jnp.float32):
    return jnp.full((L,), v, dtype)


def _sc_body(x_hbm, t_hbm, loss_hbm, prec_hbm, stats_hbm,
             x_v, t_v, lo_v, pr_v, st_v):
    wid = lax.axis_index("s") * 2 + lax.axis_index("c")
    pltpu.sync_copy(x_hbm, x_v.at[pl.ds(0, N)])
    pltpu.sync_copy(t_hbm, t_v.at[pl.ds(0, N)])
    row0 = NT + wid * RW

    def row_body(r, carry):
        loss_acc, prec_acc = carry
        row = row0 + r
        xc = x_v[pl.ds(row, L)]
        tc = t_v[pl.ds(row, L)]
        xi = jnp.full((L,), xc[0], jnp.float32)
        ti = jnp.full((L,), tc[0], jnp.int32)

        def max_body(c, m):
            xs = x_v[pl.ds(c * L, L)]
            ts = t_v[pl.ds(c * L, L)]
            sim = jnp.abs(xs - xi)
            excl = (ts == ti) & (sim >= 1.0)
            return jnp.maximum(m, jnp.where(excl, -1.0, sim))

        m = lax.fori_loop(0, NCH, max_body, jnp.full((L,), -1.0, jnp.float32),
                          unroll=2)
        thr = _splat(jnp.max(m))

        def sum_body(c, carry2):
            p_acc, s_acc = carry2
            xs = x_v[pl.ds(c * L, L)]
            ts = t_v[pl.ds(c * L, L)]
            sim = jnp.abs(xs - xi)
            pos = ts == ti
            lt1 = sim < 1.0
            below = sim < thr
            w = jnp.exp(sim * (-ALPHA))
            pn = (pos & lt1) & below
            selb = below & (jnp.logical_not(pos) | lt1)
            p_acc = p_acc + jnp.where(pn, w, 0.0)
            s_acc = s_acc + jnp.where(selb, w, 0.0)
            return p_acc, s_acc

        zero = jnp.zeros((L,), jnp.float32)
        p_acc, s_acc = lax.fori_loop(0, NCH, sum_body, (zero, zero), unroll=2)
        pv = _splat(jnp.sum(p_acc))
        sv = _splat(jnp.sum(s_acc))
        ratio = pv / sv
        loss_v = jnp.where(pv > 0.0, -_vlog(ratio), 0.0)
        loss_acc = loss_acc + loss_v
        prec_acc = prec_acc + jnp.where(loss_v < 0.6, 1.0, 0.0)
        return loss_acc, prec_acc

    zero = jnp.zeros((L,), jnp.float32)
    loss_acc, prec_acc = lax.fori_loop(0, RW, row_body, (zero, zero))

    lo_v[...] = loss_acc
    pr_v[...] = prec_acc
    pltpu.sync_copy(lo_v, loss_hbm.at[wid])
    pltpu.sync_copy(pr_v, prec_hbm.at[wid])

    @pl.when(wid == NW - 1)
    def _stats():
        xc = x_v[pl.ds(N - 1, L)]
        tc = t_v[pl.ds(N - 1, L)]
        xi = jnp.full((L,), xc[0], jnp.float32)
        ti = jnp.full((L,), tc[0], jnp.int32)

        def st_body(c, carry):
            sp, cp, sn, cn = carry
            xs = x_v[pl.ds(c * L, L)]
            ts = t_v[pl.ds(c * L, L)]
            sim = jnp.abs(xs - xi)
            pos = ts == ti
            lp = pos & (sim < 1.0)
            sp = sp + jnp.where(lp, sim, 0.0)
            cp = cp + jnp.where(lp, 1.0, 0.0)
            sn = sn + jnp.where(pos, 0.0, sim)
            cn = cn + jnp.where(pos, 0.0, 1.0)
            return sp, cp, sn, cn

        z = jnp.zeros((L,), jnp.float32)
        sp, cp, sn, cn = lax.fori_loop(0, NCH, st_body, (z, z, z, z), unroll=2)
        st_v[0, :] = _splat(jnp.sum(sp)) / _splat(jnp.sum(cp))
        st_v[1, :] = _splat(jnp.sum(sn)) / _splat(jnp.sum(cn))
        pltpu.sync_copy(st_v, stats_hbm)


_sc_call = functools.partial(
    pl.kernel,
    mesh=plsc.VectorSubcoreMesh(core_axis_name="c", subcore_axis_name="s"),
    compiler_params=pltpu.CompilerParams(needs_layout_passes=False),
    out_type=[
        jax.ShapeDtypeStruct((NW, L), jnp.float32),
        jax.ShapeDtypeStruct((NW, L), jnp.float32),
        jax.ShapeDtypeStruct((2, L), jnp.float32),
    ],
    scratch_types=[
        pltpu.VMEM((N + L,), jnp.float32),
        pltpu.VMEM((N + L,), jnp.int32),
        pltpu.VMEM((L,), jnp.float32),
        pltpu.VMEM((L,), jnp.float32),
        pltpu.VMEM((2, L), jnp.float32),
    ],
)(_sc_body)


def kernel(inputs, targets):
    t32 = targets.astype(jnp.int32)
    sc_loss, sc_prec, stats = _sc_call(inputs, t32)
    tc_loss, tc_prec = _tc_call(
        inputs.reshape(N, 1)[:NT], t32.reshape(N, 1)[:NT],
        inputs.reshape(1, N), t32.reshape(1, N))
    loss = (tc_loss[0, 0] + jnp.sum(sc_loss[:, 0])) * (1.0 / N)
    prec = (tc_prec[0, 0] + jnp.sum(sc_prec[:, 0])) * (1.0 / N)
    return (loss, prec, stats[0, 0], stats[1, 0])


# speedup vs baseline: 9.5780x; 1.1590x over previous
import jax
import jax.numpy as jnp
from jax.experimental import pallas as pl

ALPHA = 16.0
N = 4096
R = 1024
G = N // R


def _nca_body(x_row_ref, t_row_ref, x_col_ref, t_col_ref,
              loss_ref, prec_ref, mps_ref, mns_ref):
    i = pl.program_id(0)

    x_row = x_row_ref[...]
    t_row = t_row_ref[...]
    x_col = x_col_ref[...]
    t_col = t_col_ref[...]

    sim = jnp.abs(x_col - x_row)
    pos = t_col == t_row
    lt1 = sim < 1.0
    excl = pos & jnp.logical_not(lt1)
    thr = jnp.max(jnp.where(excl, -1.0, sim), axis=1, keepdims=True)

    below = sim < thr
    w = jnp.exp(-ALPHA * sim)
    pn_m = (pos & lt1) & below
    selb = below & (jnp.logical_not(pos) | lt1)
    p = jnp.sum(jnp.where(pn_m, w, 0.0), axis=1, keepdims=True)
    s = jnp.sum(jnp.where(selb, w, 0.0), axis=1, keepdims=True)
    q = s - p

    loss_i = jnp.where(p > 0.0, -jnp.log(p / (p + q)), 0.0)

    @pl.when(i == 0)
    def _init():
        loss_ref[...] = jnp.zeros_like(loss_ref)
        prec_ref[...] = jnp.zeros_like(prec_ref)

    loss_ref[...] += jnp.sum(loss_i).reshape(1, 1)
    prec_ref[...] += jnp.sum(jnp.where(loss_i < 0.6, 1.0, 0.0)).reshape(1, 1)

    @pl.when(i == G - 1)
    def _last():
        sl = sim[R - 1:R, :]
        lp = jnp.where(pos[R - 1:R, :] & lt1[R - 1:R, :], 1.0, 0.0)
        ln = jnp.where(pos[R - 1:R, :], 0.0, 1.0)
        mps_ref[...] = (jnp.sum(sl * lp) / jnp.sum(lp)).reshape(1, 1)
        mns_ref[...] = (jnp.sum(sl * ln) / jnp.sum(ln)).reshape(1, 1)
        loss_ref[...] = loss_ref[...] * (1.0 / N)
        prec_ref[...] = prec_ref[...] * (1.0 / N)


def kernel(inputs, targets):
    t32 = targets.astype(jnp.int32)
    x_rows = inputs.reshape(N, 1)
    t_rows = t32.reshape(N, 1)
    x_cols = inputs.reshape(1, N)
    t_cols = t32.reshape(1, N)

    out = pl.pallas_call(
        _nca_body,
        grid=(G,),
        in_specs=[
            pl.BlockSpec((R, 1), lambda i: (i, 0)),
            pl.BlockSpec((R, 1), lambda i: (i, 0)),
            pl.BlockSpec((1, N), lambda i: (0, 0)),
            pl.BlockSpec((1, N), lambda i: (0, 0)),
        ],
        out_specs=[
            pl.BlockSpec((1, 1), lambda i: (0, 0)),
            pl.BlockSpec((1, 1), lambda i: (0, 0)),
            pl.BlockSpec((1, 1), lambda i: (0, 0)),
            pl.BlockSpec((1, 1), lambda i: (0, 0)),
        ],
        out_shape=[jax.ShapeDtypeStruct((1, 1), jnp.float32)] * 4,
    )(x_rows, t_rows, x_cols, t_cols)

    loss, prec, mps, mns = out
    return (loss[0, 0], prec[0, 0], mps[0, 0], mns[0, 0])


# speedup vs baseline: 10.5600x; 1.1025x over previous
import jax
import jax.numpy as jnp
from jax.experimental import pallas as pl

ALPHA = 16.0
N = 4096
R = 1024
G = N // R


def _nca_body(x_row_ref, t_row_ref, x_col_ref, t_col_ref,
              loss_ref, prec_ref, mps_ref, mns_ref):
    i = pl.program_id(0)

    x_row = x_row_ref[...]
    t_row = t_row_ref[...]
    x_col = x_col_ref[...]
    t_col = t_col_ref[...]

    sim = jnp.abs(x_col - x_row)
    pos = t_col == t_row
    lt1 = sim < 1.0
    excl = pos & jnp.logical_not(lt1)
    thr = jnp.max(jnp.where(excl, -1.0, sim), axis=1, keepdims=True)

    below = sim < thr
    w = jnp.exp2(sim * (-ALPHA * 1.4426950408889634))
    pn_m = (pos & lt1) & below
    selb = below & jnp.logical_not(excl)
    p = jnp.sum(jnp.where(pn_m, w, 0.0), axis=1, keepdims=True)
    s = jnp.sum(jnp.where(selb, w, 0.0), axis=1, keepdims=True)
    q = s - p

    loss_i = jnp.where(p > 0.0, -jnp.log(p / (p + q)), 0.0)

    @pl.when(i == 0)
    def _init():
        loss_ref[...] = jnp.zeros_like(loss_ref)
        prec_ref[...] = jnp.zeros_like(prec_ref)

    loss_ref[...] += jnp.sum(loss_i).reshape(1, 1)
    prec_ref[...] += jnp.sum(jnp.where(loss_i < 0.6, 1.0, 0.0)).reshape(1, 1)

    @pl.when(i == G - 1)
    def _last():
        sl = sim[R - 1:R, :]
        lp = jnp.where(pos[R - 1:R, :] & lt1[R - 1:R, :], 1.0, 0.0)
        ln = jnp.where(pos[R - 1:R, :], 0.0, 1.0)
        mps_ref[...] = (jnp.sum(sl * lp) / jnp.sum(lp)).reshape(1, 1)
        mns_ref[...] = (jnp.sum(sl * ln) / jnp.sum(ln)).reshape(1, 1)
        loss_ref[...] = loss_ref[...] * (1.0 / N)
        prec_ref[...] = prec_ref[...] * (1.0 / N)


def kernel(inputs, targets):
    t32 = targets.astype(jnp.int32)
    x_rows = inputs.reshape(N, 1)
    t_rows = t32.reshape(N, 1)
    x_cols = inputs.reshape(1, N)
    t_cols = t32.reshape(1, N)

    out = pl.pallas_call(
        _nca_body,
        grid=(G,),
        in_specs=[
            pl.BlockSpec((R, 1), lambda i: (i, 0)),
            pl.BlockSpec((R, 1), lambda i: (i, 0)),
            pl.BlockSpec((1, N), lambda i: (0, 0)),
            pl.BlockSpec((1, N), lambda i: (0, 0)),
        ],
        out_specs=[
            pl.BlockSpec((1, 1), lambda i: (0, 0)),
            pl.BlockSpec((1, 1), lambda i: (0, 0)),
            pl.BlockSpec((1, 1), lambda i: (0, 0)),
            pl.BlockSpec((1, 1), lambda i: (0, 0)),
        ],
        out_shape=[jax.ShapeDtypeStruct((1, 1), jnp.float32)] * 4,
    )(x_rows, t_rows, x_cols, t_cols)

    loss, prec, mps, mns = out
    return (loss[0, 0], prec[0, 0], mps[0, 0], mns[0, 0])
